# Initial kernel scaffold; baseline (speedup 1.0000x reference)
import jax
import jax.numpy as jnp
from jax.experimental import pallas as pl


def kernel(edge_type_ids, embedding):
    raise NotImplementedError("write your pallas kernel here")


# speedup vs baseline: 1.3988x; 1.3988x over previous
import functools

import jax
import jax.numpy as jnp
from jax import lax
from jax.experimental import pallas as pl
from jax.experimental.pallas import tpu as pltpu
from jax.experimental.pallas import tpu_sc as plsc

_NC = 2
_NS = 16
_NW = _NC * _NS

_CHUNK = 40


def _make_sc_gather(n_rows, n_types, d):
  b_per_w = n_rows // _NW
  n_chunk = b_per_w // _CHUNK
  mesh = plsc.VectorSubcoreMesh(core_axis_name="c", subcore_axis_name="s")

  @functools.partial(
      pl.kernel,
      out_type=jax.ShapeDtypeStruct((n_rows, d), jnp.float32),
      mesh=mesh,
      scratch_types=[
          pltpu.VMEM((n_chunk, _CHUNK), jnp.int32),
          pltpu.VMEM((_CHUNK, d), jnp.float32),
          pltpu.SemaphoreType.DMA,
      ],
  )
  def k(ids_hbm, table_hbm, out_hbm, idx_v, rows_v, sem):
    wid = lax.axis_index("s") * _NC + lax.axis_index("c")
    pltpu.sync_copy(ids_hbm.at[wid], idx_v)
    base = wid * b_per_w

    def body(j, carry):
      start = pl.multiple_of(base + j * _CHUNK, 8)
      pltpu.async_copy(table_hbm.at[idx_v.at[j]], rows_v, sem).wait()
      pltpu.sync_copy(rows_v, out_hbm.at[pl.ds(start, _CHUNK)])
      return carry

    lax.fori_loop(0, n_chunk, body, 0)

  return k


def kernel(edge_type_ids, embedding):
  orig_shape = edge_type_ids.shape
  n_types, d = embedding.shape
  flat = edge_type_ids.reshape(-1)
  n = flat.shape[0]

  per = _NW * _CHUNK
  n_pad = (-n) % per
  if n_pad:
    flat = jnp.concatenate([flat, jnp.zeros((n_pad,), jnp.int32)])
  total = n + n_pad
  ids3d = flat.reshape(_NW, total // (_NW * _CHUNK), _CHUNK)
  ids3d = ids3d + (n_types * jnp.arange(_NW, dtype=jnp.int32))[:, None, None]
  table_rep = jnp.tile(embedding, (_NW, 1))

  out = _make_sc_gather(total, n_types, d)(ids3d, table_rep)
  if n_pad:
    out = out[:n]
  return out.reshape(*orig_shape, d)


# speedup vs baseline: 1.5746x; 1.1257x over previous
import functools

import jax
import jax.numpy as jnp
from jax import lax
from jax.experimental import pallas as pl
from jax.experimental.pallas import tpu as pltpu
from jax.experimental.pallas import tpu_sc as plsc

_NC = 2
_NS = 16
_NW = _NC * _NS

_CHUNK = 40


def _make_sc_gather(n_rows, n_types, d):
  b_per_w = n_rows // _NW
  n_chunk = b_per_w // _CHUNK
  mesh = plsc.VectorSubcoreMesh(core_axis_name="c", subcore_axis_name="s")

  @functools.partial(
      pl.kernel,
      out_type=jax.ShapeDtypeStruct((n_rows, d), jnp.float32),
      mesh=mesh,
      scratch_types=[
          pltpu.VMEM((n_chunk, _CHUNK), jnp.int32),
          pltpu.VMEM((2, _CHUNK, d), jnp.float32),
          pltpu.SemaphoreType.DMA,
          pltpu.SemaphoreType.DMA,
      ],
  )
  def k(ids_hbm, table_hbm, out_hbm, idx_v, rows_v, gsem, osem):
    wid = lax.axis_index("s") * _NC + lax.axis_index("c")
    pltpu.sync_copy(ids_hbm.at[wid], idx_v)
    base = wid * b_per_w

    pltpu.async_copy(table_hbm.at[idx_v.at[0]], rows_v.at[0], gsem)

    def body(j, carry):
      slot = lax.rem(j, 2)
      nslot = lax.rem(j + 1, 2)
      start = pl.multiple_of(base + j * _CHUNK, 8)
      pltpu.make_async_copy(
          table_hbm.at[idx_v.at[j]], rows_v.at[slot], gsem).wait()

      @pl.when(j > 0)
      def _():
        prev = pl.multiple_of(base + (j - 1) * _CHUNK, 8)
        pltpu.make_async_copy(
            rows_v.at[nslot], out_hbm.at[pl.ds(prev, _CHUNK)], osem).wait()

      pltpu.async_copy(
          rows_v.at[slot], out_hbm.at[pl.ds(start, _CHUNK)], osem)

      @pl.when(j + 1 < n_chunk)
      def _():
        pltpu.async_copy(
            table_hbm.at[idx_v.at[j + 1]], rows_v.at[nslot], gsem)

      return carry

    lax.fori_loop(0, n_chunk, body, 0)
    last = pl.multiple_of(base + (n_chunk - 1) * _CHUNK, 8)
    pltpu.make_async_copy(
        rows_v.at[(n_chunk - 1) % 2],
        out_hbm.at[pl.ds(last, _CHUNK)], osem).wait()

  return k


def kernel(edge_type_ids, embedding):
  orig_shape = edge_type_ids.shape
  n_types, d = embedding.shape
  flat = edge_type_ids.reshape(-1)
  n = flat.shape[0]

  per = _NW * _CHUNK
  n_pad = (-n) % per
  if n_pad:
    flat = jnp.concatenate([flat, jnp.zeros((n_pad,), jnp.int32)])
  total = n + n_pad
  ids3d = flat.reshape(_NW, total // (_NW * _CHUNK), _CHUNK)
  ids3d = ids3d + (n_types * jnp.arange(_NW, dtype=jnp.int32))[:, None, None]
  table_rep = jnp.tile(embedding, (_NW, 1))

  out = _make_sc_gather(total, n_types, d)(ids3d, table_rep)
  if n_pad:
    out = out[:n]
  return out.reshape(*orig_shape, d)
